# baseline (speedup 1.0000x reference)
import functools
import jax
import jax.numpy as jnp
from jax import lax
from jax.experimental import pallas as pl
from jax.experimental.pallas import tpu as pltpu
from jax.experimental.pallas import tpu_sc as plsc

D = 64
SCALE = 8.0
HALVES = ((0, 104), (104, 96))
NC, NS = 2, 16
NW = NC * NS
NBUF = 3
TC = 336
NCHUNK = 93
VMAIN = 31248


def _transpose_table(table_t):
    V = table_t.shape[1]

    mesh = plsc.VectorSubcoreMesh(
        core_axis_name="c", subcore_axis_name="s",
        num_cores=NC, num_subcores=NS)

    @functools.partial(
        pl.kernel,
        out_type=jax.ShapeDtypeStruct((V, D), jnp.float32),
        mesh=mesh,
        compiler_params=pltpu.CompilerParams(
            use_tc_tiling_on_sc=False, skip_device_barrier=True,
            needs_layout_passes=False),
        scratch_types=[
            pltpu.VMEM((D, TC), jnp.float32),
            pltpu.VMEM((D, TC), jnp.float32),
            pltpu.VMEM((TC, D), jnp.float32),
            pltpu.VMEM((TC, D), jnp.float32),
            pltpu.SemaphoreType.DMA,
            pltpu.SemaphoreType.DMA,
            pltpu.SemaphoreType.DMA,
            pltpu.SemaphoreType.DMA,
        ],
    )
    def tr(tt_hbm, out_hbm, in0, in1, out0, out1,
           isem0, isem1, osem0, osem1):
        ins = (in0, in1)
        outs = (out0, out1)
        isems = (isem0, isem1)
        osems = (osem0, osem1)
        wid = lax.axis_index("s") * NC + lax.axis_index("c")
        vbase = wid * VMAIN

        def start_in(g, s):
            pltpu.async_copy(
                tt_hbm.at[pl.ds(0, D), pl.ds(vbase + g * TC, TC)],
                ins[s], isems[s])

        def wait_in(s):
            pltpu.make_async_copy(
                tt_hbm.at[pl.ds(0, D), pl.ds(0, TC)],
                ins[s], isems[s]).wait()

        def start_out(g, s):
            pltpu.async_copy(
                outs[s], out_hbm.at[pl.ds(vbase + g * TC, TC)],
                osems[s])

        def wait_out(s):
            pltpu.make_async_copy(
                outs[s], out_hbm.at[pl.ds(0, TC)], osems[s]).wait()

        row_ids = [lax.iota(jnp.int32, 16) + k * 16 for k in range(D // 16)]

        def transpose(s):
            def col_body(v, c):
                col = jnp.full((16,), v, jnp.int32)
                for k in range(D // 16):
                    vals = plsc.load_gather(ins[s], [row_ids[k], col])
                    outs[s][v, pl.ds(k * 16, 16)] = vals
                return c
            lax.fori_loop(0, TC, col_body, 0)

        def step(g, s, do_issue, do_out_wait):
            wait_in(s)
            if do_out_wait:
                wait_out(s)
            transpose(s)
            start_out(g, s)
            if do_issue:
                start_in(g + 2, s)

        start_in(0, 0)
        start_in(1, 1)
        step(0, 0, True, False)
        step(1, 1, True, False)
        def main_body(g, carry):
            step(2 * g, 0, True, True)
            step(2 * g + 1, 1, True, True)
            return carry
        lax.fori_loop(1, 45, main_body, 0)
        step(90, 0, True, True)
        step(91, 1, False, True)
        step(92, 0, False, True)
        wait_out(1)
        wait_out(0)

        @pl.when(wid == NW - 1)
        def _():
            tail = NW * VMAIN
            pltpu.sync_copy(
                tt_hbm.at[pl.ds(0, D), pl.ds(tail, D)],
                in0.at[pl.ds(0, D), pl.ds(0, D)])
            def col_body(v, c):
                col = jnp.full((16,), v, jnp.int32)
                for k in range(D // 16):
                    vals = plsc.load_gather(in0, [row_ids[k], col])
                    out0[v, pl.ds(k * 16, 16)] = vals
                return c
            lax.fori_loop(0, D, col_body, 0)
            pltpu.sync_copy(out0.at[pl.ds(0, D)],
                            out_hbm.at[pl.ds(tail, D)])

    return tr(table_t)


def _gather_scaled(tok2d, table):
    BATCH, SEQ = tok2d.shape
    rows_per_w = BATCH // NW

    mesh = plsc.VectorSubcoreMesh(
        core_axis_name="c", subcore_axis_name="s",
        num_cores=NC, num_subcores=NS)

    @functools.partial(
        pl.kernel,
        out_type=jax.ShapeDtypeStruct((BATCH, SEQ, D), jnp.float32),
        mesh=mesh,
        compiler_params=pltpu.CompilerParams(
            use_tc_tiling_on_sc=False, skip_device_barrier=True),
        scratch_types=[
            pltpu.VMEM((rows_per_w, SEQ), jnp.int32),
            pltpu.VMEM((NBUF, SEQ, D), jnp.float32),
            pltpu.VMEM((NBUF, SEQ, D), jnp.float32),
            pltpu.SemaphoreType.DMA,
            pltpu.SemaphoreType.DMA,
            pltpu.SemaphoreType.DMA,
            pltpu.SemaphoreType.DMA,
            pltpu.SemaphoreType.DMA,
            pltpu.SemaphoreType.DMA,
        ],
    )
    def emb(tok_hbm, table_hbm, out_hbm, idx_v, raw_v, outb_v,
            gsem0, gsem1, gsem2, ssem0, ssem1, ssem2):
        gsems = (gsem0, gsem1, gsem2)
        ssems = (ssem0, ssem1, ssem2)
        wid = lax.axis_index("s") * NC + lax.axis_index("c")
        brow = wid * rows_per_w

        pltpu.sync_copy(tok_hbm.at[pl.ds(brow, rows_per_w)], idx_v)

        def start_gathers(r, s):
            for off, ln in HALVES:
                pltpu.async_copy(
                    table_hbm.at[idx_v.at[r, pl.ds(off, ln)]],
                    raw_v.at[s, pl.ds(off, ln)],
                    gsems[s])

        def wait_gathers(s):
            for off, ln in HALVES:
                pltpu.make_async_copy(
                    table_hbm.at[pl.ds(0, ln)],
                    raw_v.at[s, pl.ds(off, ln)],
                    gsems[s]).wait()

        def start_store(r, s):
            pltpu.async_copy(outb_v.at[s], out_hbm.at[brow + r], ssems[s])

        def wait_store(s):
            pltpu.make_async_copy(outb_v.at[s], out_hbm.at[brow],
                                  ssems[s]).wait()

        def scale(s):
            def row_body(i, c):
                for k in range(D // 16):
                    sl = pl.ds(k * 16, 16)
                    outb_v[s, i, sl] = raw_v[s, i, sl] * SCALE
                return c
            lax.fori_loop(0, SEQ, row_body, 0)

        def step(r, s, do_issue, do_store_wait):
            wait_gathers(s)
            if do_store_wait:
                wait_store(s)
            scale(s)
            if do_issue:
                start_gathers(r + NBUF, s)
            start_store(r, s)

        for s in range(NBUF):
            start_gathers(s, s)
        for r in range(NBUF):
            step(r, r % NBUF, True, False)

        n_main = (rows_per_w - 2 * NBUF) // NBUF
        def main_body(g, carry):
            for b in range(NBUF):
                step(g * NBUF + b, b, True, True)
            return carry
        lax.fori_loop(1, 1 + n_main, main_body, 0)

        done = NBUF + n_main * NBUF
        for r in range(done, rows_per_w):
            step(r, r % NBUF, r + NBUF < rows_per_w, True)

        for s in range(NBUF):
            wait_store(s)

    return emb(tok2d, table)


def kernel(token, embeddings):
    tok2d = token.astype(jnp.int32)
    table = _transpose_table(embeddings.T)
    return _gather_scaled(tok2d, table)


# speedup vs baseline: 6.4645x; 6.4645x over previous
import functools
import jax
import jax.numpy as jnp
from jax import lax
from jax.experimental import pallas as pl
from jax.experimental.pallas import tpu as pltpu
from jax.experimental.pallas import tpu_sc as plsc

D = 64
DP = 128
SCALE = 8.0
HALVES = ((0, 104), (104, 96))
NC, NS = 2, 16
NW = NC * NS
NBUF = 3


def kernel(token, embeddings):
    BATCH, SEQ = token.shape
    rows_per_w = BATCH // NW
    tok2d = token.astype(jnp.int32)
    tblp = jnp.pad(embeddings, ((0, 0), (0, DP - D)))

    mesh = plsc.VectorSubcoreMesh(
        core_axis_name="c", subcore_axis_name="s",
        num_cores=NC, num_subcores=NS)

    @functools.partial(
        pl.kernel,
        out_type=jax.ShapeDtypeStruct((BATCH, SEQ, DP), jnp.float32),
        mesh=mesh,
        compiler_params=pltpu.CompilerParams(
            use_tc_tiling_on_sc=False, skip_device_barrier=True),
        scratch_types=[
            pltpu.VMEM((rows_per_w, SEQ), jnp.int32),
            pltpu.VMEM((NBUF, SEQ, DP), jnp.float32),
            pltpu.SemaphoreType.DMA,
            pltpu.SemaphoreType.DMA,
            pltpu.SemaphoreType.DMA,
            pltpu.SemaphoreType.DMA,
            pltpu.SemaphoreType.DMA,
            pltpu.SemaphoreType.DMA,
        ],
    )
    def emb(tok_hbm, table_hbm, out_hbm, idx_v, raw_v,
            gsem0, gsem1, gsem2, ssem0, ssem1, ssem2):
        gsems = (gsem0, gsem1, gsem2)
        ssems = (ssem0, ssem1, ssem2)
        wid = lax.axis_index("s") * NC + lax.axis_index("c")
        brow = wid * rows_per_w

        pltpu.sync_copy(tok_hbm.at[pl.ds(brow, rows_per_w)], idx_v)

        def start_gathers(r, s):
            for off, ln in HALVES:
                pltpu.async_copy(
                    table_hbm.at[idx_v.at[r, pl.ds(off, ln)]],
                    raw_v.at[s, pl.ds(off, ln)],
                    gsems[s])

        def wait_gathers(s):
            for off, ln in HALVES:
                pltpu.make_async_copy(
                    table_hbm.at[pl.ds(0, ln)],
                    raw_v.at[s, pl.ds(off, ln)],
                    gsems[s]).wait()

        def start_store(r, s):
            pltpu.async_copy(raw_v.at[s], out_hbm.at[brow + r], ssems[s])

        def wait_store(s):
            pltpu.make_async_copy(raw_v.at[s], out_hbm.at[brow],
                                  ssems[s]).wait()

        def scale(s):
            def row_body(i, c):
                for k in range(D // 16):
                    sl = pl.ds(k * 16, 16)
                    raw_v[s, i, sl] = raw_v[s, i, sl] * SCALE
                return c
            lax.fori_loop(0, SEQ, row_body, 0)

        def step(r, s, do_issue):
            wait_gathers(s)
            scale(s)
            start_store(r, s)
            if do_issue:
                wait_store(s)
                start_gathers(r + NBUF, s)

        for s in range(NBUF):
            start_gathers(s, s)
        for r in range(NBUF):
            step(r, r % NBUF, True)

        n_main = (rows_per_w - 2 * NBUF) // NBUF
        def main_body(g, carry):
            for b in range(NBUF):
                step(g * NBUF + b, b, True)
            return carry
        lax.fori_loop(1, 1 + n_main, main_body, 0)

        done = NBUF + n_main * NBUF
        for r in range(done, rows_per_w):
            step(r, r % NBUF, r + NBUF < rows_per_w)

        for s in range(NBUF):
            wait_store(s)

    out3 = emb(tok2d, tblp)
    return out3[:, :, :D]
